# baseline (speedup 1.0000x reference)
import functools

import jax
import jax.numpy as jnp
from jax import lax
from jax.experimental import pallas as pl
from jax.experimental.pallas import tpu as pltpu
from jax.experimental.pallas import tpu_sc as plsc

_NC = 2
_NS = 16
_NW = _NC * _NS

_B = 1024
_T = 20
_V = 35
_E = 64
_ROWS = _T * _V
_RPAD = 768
_CHUNK = 128
_NCHUNK = _RPAD // _CHUNK
_OPAD = 704


def _sc_gather_fn():
    spw = _B // _NW

    mesh = plsc.VectorSubcoreMesh(
        core_axis_name="c", subcore_axis_name="s",
        num_cores=_NC, num_subcores=_NS)

    @functools.partial(
        pl.kernel,
        out_type=jax.ShapeDtypeStruct((_B * _OPAD, 128), jnp.float32),
        mesh=mesh,
        scratch_types=[
            pltpu.VMEM((_NCHUNK, _CHUNK), jnp.int32),
            pltpu.VMEM((_RPAD, 128), jnp.float32),
            pltpu.SemaphoreType.DMA,
        ],
        compiler_params=pltpu.CompilerParams(use_tc_tiling_on_sc=False),
    )
    def sc_gather(idx_hbm, table_hbm, out_hbm, idx_v, y_v, gsem):
        wid = lax.axis_index("s") * _NC + lax.axis_index("c")
        base = wid * spw

        def body(i, carry):
            bb = base + i
            pltpu.sync_copy(idx_hbm.at[pl.ds(bb * _NCHUNK, _NCHUNK)], idx_v)
            cps = []
            for ch in range(_NCHUNK):
                cps.append(pltpu.async_copy(
                    table_hbm.at[idx_v.at[ch]],
                    y_v.at[pl.ds(ch * _CHUNK, _CHUNK)], gsem))
            for cp in cps:
                cp.wait()
            pltpu.sync_copy(y_v.at[pl.ds(0, _OPAD)],
                            out_hbm.at[pl.ds(bb * _OPAD, _OPAD)])
            return carry

        lax.fori_loop(0, spw, body, 0)

    return sc_gather


_sc_gather_cache = []


def _sc_gather(idx, table):
    if not _sc_gather_cache:
        _sc_gather_cache.append(_sc_gather_fn())
    return _sc_gather_cache[0](idx, table)


def _norm_body(g_ref, pe_ref, o_ref):
    g = g_ref[...].reshape(-1, _OPAD, 128)[:, 0:_ROWS, :]
    g = g.reshape(-1, _T, _V, 128)
    y = jnp.concatenate(
        [g[:, :, 0:_V - 1, 0:_E], g[:, :, _V - 1:_V, _E:2 * _E]], axis=2)
    y = y + pe_ref[...][None, :, None, :]
    m = jnp.mean(y, axis=(1, 2, 3), keepdims=True)
    d = y - m
    v = jnp.mean(d * d, axis=(1, 2, 3), keepdims=True)
    o_ref[...] = d * lax.rsqrt(v + 1e-5)


def _norm_fn():
    bb = 8
    return pl.pallas_call(
        _norm_body,
        grid=(_B // bb,),
        in_specs=[
            pl.BlockSpec((bb * _OPAD, 128), lambda i: (i, 0)),
            pl.BlockSpec((_T, _E), lambda i: (0, 0)),
        ],
        out_specs=pl.BlockSpec((bb, _T, _V, _E), lambda i: (i, 0, 0, 0)),
        out_shape=jax.ShapeDtypeStruct((_B, _T, _V, _E), jnp.float32),
    )


_norm = _norm_fn()


def kernel(diag_seq, proc_seq, drug_seq, delta_t, service, admtype, insur,
           marit, seq_length, token_table, pe_dt, pe_pos):
    b, t = delta_t.shape
    vocab, e = token_table.shape

    dt = delta_t / 15.0
    len_mask = jnp.arange(t)[None, :] < seq_length[:, None]
    dt = jnp.cumsum(jnp.round(dt), axis=1) * len_mask.astype(dt.dtype)
    dt_idx = jnp.clip(dt.astype(jnp.int32), 0, pe_dt.shape[0] - 1)

    tok34 = jnp.concatenate(
        [diag_seq, proc_seq, drug_seq, service, admtype,
         insur[..., None], marit[..., None]], axis=2)
    idx35 = jnp.concatenate([tok34, dt_idx[..., None]], axis=2)
    idx = jnp.pad(idx35.reshape(b, _ROWS), ((0, 0), (0, _RPAD - _ROWS)))
    idx = idx.reshape(b * _NCHUNK, _CHUNK)

    table128 = jnp.concatenate([token_table, pe_dt], axis=1)

    g = _sc_gather(idx, table128)

    return _norm(g, pe_pos[:t])


# speedup vs baseline: 3.3324x; 3.3324x over previous
import functools

import jax
import jax.numpy as jnp
from jax import lax
from jax.experimental import pallas as pl
from jax.experimental.pallas import tpu as pltpu
from jax.experimental.pallas import tpu_sc as plsc

_NC = 2
_NS = 16
_NW = _NC * _NS

_B = 1024
_T = 20
_V = 35
_E = 64
_ROWS = _T * _V
_RPAD = 704
_CHUNK = 128
_N_ELT = _ROWS * _E
_EPS = 1e-5


def _sc_fused_fn():
    spw = _B // _NW
    nfull, rem = divmod(_RPAD, _CHUNK)

    mesh = plsc.VectorSubcoreMesh(
        core_axis_name="c", subcore_axis_name="s",
        num_cores=_NC, num_subcores=_NS)

    @functools.partial(
        pl.kernel,
        out_type=jax.ShapeDtypeStruct((_B, _T, _V, _E), jnp.float32),
        mesh=mesh,
        scratch_types=[
            pltpu.VMEM((_RPAD,), jnp.int32),
            pltpu.VMEM((_RPAD, _E), jnp.float32),
            pltpu.VMEM((24, _E), jnp.float32),
            pltpu.SemaphoreType.DMA,
            pltpu.SemaphoreType.DMA,
        ],
        compiler_params=pltpu.CompilerParams(
            use_tc_tiling_on_sc=False, needs_layout_passes=False),
    )
    def sc_fused(idx_hbm, table_hbm, pe_hbm, out_hbm, idx_v, y_v, pe_v,
                 gsem, osem):
        wid = lax.axis_index("s") * _NC + lax.axis_index("c")
        base = wid * spw
        zero16 = jnp.full((16,), 0.0, jnp.float32)
        pltpu.sync_copy(pe_hbm, pe_v)

        def body(i, carry):
            bb = base + i
            pltpu.sync_copy(idx_hbm.at[bb], idx_v)
            cps = []
            for ch in range(nfull):
                cps.append(pltpu.async_copy(
                    table_hbm.at[idx_v.at[pl.ds(ch * _CHUNK, _CHUNK)]],
                    y_v.at[pl.ds(ch * _CHUNK, _CHUNK)], gsem))
            if rem:
                cps.append(pltpu.async_copy(
                    table_hbm.at[idx_v.at[pl.ds(nfull * _CHUNK, rem)]],
                    y_v.at[pl.ds(nfull * _CHUNK, rem)], gsem))
            for cp in cps:
                cp.wait()

            def t_acc(tt, acc):
                pe4 = [pe_v[tt, pl.ds(k * 16, 16)] for k in range(4)]

                def v_acc(vv, acc2):
                    r = tt * _V + vv
                    out = list(acc2)
                    for k in range(4):
                        x = y_v[r, pl.ds(k * 16, 16)] + pe4[k]
                        y_v[r, pl.ds(k * 16, 16)] = x
                        out[k] = out[k] + x
                        out[4 + k] = out[4 + k] + x * x
                    return tuple(out)

                return lax.fori_loop(0, _V, v_acc, acc)

            acc = lax.fori_loop(0, _T, t_acc, (zero16,) * 8)
            s16 = acc[0] + acc[1] + acc[2] + acc[3]
            q16 = acc[4] + acc[5] + acc[6] + acc[7]
            mean = jnp.sum(s16) * (1.0 / _N_ELT)
            var = jnp.sum(q16) * (1.0 / _N_ELT) - mean * mean
            ve = var + _EPS
            bits = lax.bitcast_convert_type(ve, jnp.int32)
            bits = 0x5F3759DF - lax.shift_right_arithmetic(bits, 1)
            r0 = lax.bitcast_convert_type(bits, jnp.float32)
            for _ in range(4):
                r0 = r0 * (1.5 - 0.5 * ve * r0 * r0)
            inv16 = jnp.full((16,), r0, jnp.float32)
            mv16 = jnp.full((16,), mean * r0, jnp.float32)

            def r_norm(r, c):
                for k in range(4):
                    y_v[r, pl.ds(k * 16, 16)] = (
                        y_v[r, pl.ds(k * 16, 16)] * inv16 - mv16)
                return c

            lax.fori_loop(0, _ROWS, r_norm, 0)

            ocps = []
            for tt in range(_T):
                ocps.append(pltpu.async_copy(
                    y_v.at[pl.ds(tt * _V, _V)], out_hbm.at[bb, tt], osem))
            for cp in ocps:
                cp.wait()
            return carry

        lax.fori_loop(0, spw, body, 0)

    return sc_fused


_sc_fused_cache = []


def _sc_fused(idx, table, pe):
    if not _sc_fused_cache:
        _sc_fused_cache.append(_sc_fused_fn())
    return _sc_fused_cache[0](idx, table, pe)


def kernel(diag_seq, proc_seq, drug_seq, delta_t, service, admtype, insur,
           marit, seq_length, token_table, pe_dt, pe_pos):
    b, t = delta_t.shape
    vocab, e = token_table.shape

    dt = delta_t / 15.0
    len_mask = jnp.arange(t)[None, :] < seq_length[:, None]
    dt = jnp.cumsum(jnp.round(dt), axis=1) * len_mask.astype(dt.dtype)
    dt_idx = jnp.clip(dt.astype(jnp.int32), 0, pe_dt.shape[0] - 1)

    tok34 = jnp.concatenate(
        [diag_seq, proc_seq, drug_seq, service, admtype,
         insur[..., None], marit[..., None]], axis=2)
    idx35 = jnp.concatenate([tok34, (dt_idx + vocab)[..., None]], axis=2)
    idx = jnp.pad(idx35.reshape(b, _ROWS), ((0, 0), (0, _RPAD - _ROWS)))

    table64 = jnp.concatenate([token_table, pe_dt], axis=0)
    pe24 = jnp.pad(pe_pos[:t], ((0, 4), (0, 0)))

    return _sc_fused(idx, table64, pe24)


# speedup vs baseline: 3.5888x; 1.0770x over previous
import functools

import jax
import jax.numpy as jnp
from jax import lax
from jax.experimental import pallas as pl
from jax.experimental.pallas import tpu as pltpu
from jax.experimental.pallas import tpu_sc as plsc

_NC = 2
_NS = 16
_NW = _NC * _NS

_B = 1024
_T = 20
_V = 35
_E = 64
_ROWS = _T * _V
_RPAD = 704
_CHUNK = 128
_N_ELT = _ROWS * _E
_EPS = 1e-5


def _sc_fused_fn():
    spw = _B // _NW
    nfull, rem = divmod(_RPAD, _CHUNK)

    mesh = plsc.VectorSubcoreMesh(
        core_axis_name="c", subcore_axis_name="s",
        num_cores=_NC, num_subcores=_NS)

    @functools.partial(
        pl.kernel,
        out_type=jax.ShapeDtypeStruct((_B, _T, _V, _E), jnp.float32),
        mesh=mesh,
        scratch_types=[
            pltpu.VMEM((_RPAD,), jnp.int32),
            pltpu.VMEM((_RPAD,), jnp.int32),
            pltpu.VMEM((_RPAD, _E), jnp.float32),
            pltpu.VMEM((_RPAD, _E), jnp.float32),
            pltpu.VMEM((24, _E), jnp.float32),
            pltpu.SemaphoreType.DMA,
            pltpu.SemaphoreType.DMA,
            pltpu.SemaphoreType.DMA,
            pltpu.SemaphoreType.DMA,
        ],
        compiler_params=pltpu.CompilerParams(
            use_tc_tiling_on_sc=False, needs_layout_passes=False),
    )
    def sc_fused(idx_hbm, table_hbm, pe_hbm, out_hbm, idx0, idx1, y0, y1,
                 pe_v, g0, g1, o0, o1):
        wid = lax.axis_index("s") * _NC + lax.axis_index("c")
        base = wid * spw
        zero16 = jnp.full((16,), 0.0, jnp.float32)
        pltpu.sync_copy(pe_hbm, pe_v)
        idxs = (idx0, idx1)
        ys = (y0, y1)
        gs = (g0, g1)
        os_ = (o0, o1)

        def gcps(idx_v, y_v, gsem):
            cps = []
            for ch in range(nfull):
                cps.append(pltpu.make_async_copy(
                    table_hbm.at[idx_v.at[pl.ds(ch * _CHUNK, _CHUNK)]],
                    y_v.at[pl.ds(ch * _CHUNK, _CHUNK)], gsem))
            if rem:
                cps.append(pltpu.make_async_copy(
                    table_hbm.at[idx_v.at[pl.ds(nfull * _CHUNK, rem)]],
                    y_v.at[pl.ds(nfull * _CHUNK, rem)], gsem))
            return cps

        def ocps(bb, y_v, osem):
            return [pltpu.make_async_copy(
                y_v.at[pl.ds(tt * _V, _V)], out_hbm.at[bb, tt], osem)
                for tt in range(_T)]

        def process(i, p):
            bb = base + i
            idx_v, y_v = idxs[p], ys[p]
            for cp in gcps(idx_v, y_v, gs[p]):
                cp.wait()

            @pl.when(i > 0)
            def _():
                for cp in ocps(bb - 1, ys[1 - p], os_[1 - p]):
                    cp.wait()

            @pl.when(i + 1 < spw)
            def _():
                pltpu.sync_copy(idx_hbm.at[bb + 1], idxs[1 - p])
                for cp in gcps(idxs[1 - p], ys[1 - p], gs[1 - p]):
                    cp.start()

            def t_acc(tt, acc):
                pe4 = [pe_v[tt, pl.ds(k * 16, 16)] for k in range(4)]

                def v_acc(vv, acc2):
                    r = tt * _V + vv
                    out = list(acc2)
                    for k in range(4):
                        x = y_v[r, pl.ds(k * 16, 16)] + pe4[k]
                        y_v[r, pl.ds(k * 16, 16)] = x
                        out[k] = out[k] + x
                        out[4 + k] = out[4 + k] + x * x
                    return tuple(out)

                return lax.fori_loop(0, _V, v_acc, acc)

            acc = lax.fori_loop(0, _T, t_acc, (zero16,) * 8)
            s16 = acc[0] + acc[1] + acc[2] + acc[3]
            q16 = acc[4] + acc[5] + acc[6] + acc[7]
            mean = jnp.sum(s16) * (1.0 / _N_ELT)
            var = jnp.sum(q16) * (1.0 / _N_ELT) - mean * mean
            ve = var + _EPS
            bits = lax.bitcast_convert_type(ve, jnp.int32)
            bits = 0x5F3759DF - lax.shift_right_arithmetic(bits, 1)
            r0 = lax.bitcast_convert_type(bits, jnp.float32)
            for _ in range(4):
                r0 = r0 * (1.5 - 0.5 * ve * r0 * r0)
            inv16 = jnp.full((16,), r0, jnp.float32)
            mv16 = jnp.full((16,), mean * r0, jnp.float32)

            def r_norm(r, c):
                for k in range(4):
                    y_v[r, pl.ds(k * 16, 16)] = (
                        y_v[r, pl.ds(k * 16, 16)] * inv16 - mv16)
                return c

            lax.fori_loop(0, _ROWS, r_norm, 0)

            for cp in ocps(bb, y_v, os_[p]):
                cp.start()

        def body2(ii, carry):
            process(2 * ii, 0)
            process(2 * ii + 1, 1)
            return carry

        pltpu.sync_copy(idx_hbm.at[base], idx0)
        for cp in gcps(idx0, y0, g0):
            cp.start()
        lax.fori_loop(0, spw // 2, body2, 0)
        for cp in ocps(base + spw - 1, y1, o1):
            cp.wait()

    return sc_fused


_sc_fused_cache = []


def _sc_fused(idx, table, pe):
    if not _sc_fused_cache:
        _sc_fused_cache.append(_sc_fused_fn())
    return _sc_fused_cache[0](idx, table, pe)


def kernel(diag_seq, proc_seq, drug_seq, delta_t, service, admtype, insur,
           marit, seq_length, token_table, pe_dt, pe_pos):
    b, t = delta_t.shape
    vocab, e = token_table.shape

    dt = delta_t / 15.0
    len_mask = jnp.arange(t)[None, :] < seq_length[:, None]
    dt = jnp.cumsum(jnp.round(dt), axis=1) * len_mask.astype(dt.dtype)
    dt_idx = jnp.clip(dt.astype(jnp.int32), 0, pe_dt.shape[0] - 1)

    tok34 = jnp.concatenate(
        [diag_seq, proc_seq, drug_seq, service, admtype,
         insur[..., None], marit[..., None]], axis=2)
    idx35 = jnp.concatenate([tok34, (dt_idx + vocab)[..., None]], axis=2)
    idx = jnp.pad(idx35.reshape(b, _ROWS), ((0, 0), (0, _RPAD - _ROWS)))

    table64 = jnp.concatenate([token_table, pe_dt], axis=0)
    pe24 = jnp.pad(pe_pos[:t], ((0, 4), (0, 0)))

    return _sc_fused(idx, table64, pe24)


# speedup vs baseline: 3.6187x; 1.0083x over previous
import functools

import jax
import jax.numpy as jnp
from jax import lax
from jax.experimental import pallas as pl
from jax.experimental.pallas import tpu as pltpu
from jax.experimental.pallas import tpu_sc as plsc

_NC = 2
_NS = 16
_NW = _NC * _NS

_B = 1024
_T = 20
_V = 35
_E = 64
_ROWS = _T * _V
_RPAD = 704
_CHUNK = 128
_N_ELT = _ROWS * _E
_EPS = 1e-5


def _sc_fused_fn():
    spw = _B // _NW
    nfull, rem = divmod(_RPAD, _CHUNK)

    mesh = plsc.VectorSubcoreMesh(
        core_axis_name="c", subcore_axis_name="s",
        num_cores=_NC, num_subcores=_NS)

    @functools.partial(
        pl.kernel,
        out_type=jax.ShapeDtypeStruct((_B, _T, _V, _E), jnp.float32),
        mesh=mesh,
        scratch_types=[
            pltpu.VMEM((_RPAD,), jnp.int32),
            pltpu.VMEM((_RPAD,), jnp.int32),
            pltpu.VMEM((_RPAD, _E), jnp.float32),
            pltpu.VMEM((_RPAD, _E), jnp.float32),
            pltpu.VMEM((24, _E), jnp.float32),
            pltpu.SemaphoreType.DMA,
            pltpu.SemaphoreType.DMA,
            pltpu.SemaphoreType.DMA,
            pltpu.SemaphoreType.DMA,
        ],
        compiler_params=pltpu.CompilerParams(
            use_tc_tiling_on_sc=False, needs_layout_passes=False),
    )
    def sc_fused(idx_hbm, table_hbm, pe_hbm, out_hbm, idx0, idx1, y0, y1,
                 pe_v, g0, g1, o0, o1):
        wid = lax.axis_index("s") * _NC + lax.axis_index("c")
        base = wid * spw
        zero16 = jnp.full((16,), 0.0, jnp.float32)
        pltpu.sync_copy(pe_hbm, pe_v)
        idxs = (idx0, idx1)
        ys = (y0, y1)
        gs = (g0, g1)
        os_ = (o0, o1)

        def gcps(idx_v, y_v, gsem):
            cps = []
            for ch in range(nfull):
                cps.append(pltpu.make_async_copy(
                    table_hbm.at[idx_v.at[pl.ds(ch * _CHUNK, _CHUNK)]],
                    y_v.at[pl.ds(ch * _CHUNK, _CHUNK)], gsem))
            if rem:
                cps.append(pltpu.make_async_copy(
                    table_hbm.at[idx_v.at[pl.ds(nfull * _CHUNK, rem)]],
                    y_v.at[pl.ds(nfull * _CHUNK, rem)], gsem))
            return cps

        def ocps(bb, y_v, osem):
            return [pltpu.make_async_copy(
                y_v.at[pl.ds(tt * _V, _V)], out_hbm.at[bb, tt], osem)
                for tt in range(_T)]

        def process(i, p):
            bb = base + i
            idx_v, y_v = idxs[p], ys[p]
            for cp in gcps(idx_v, y_v, gs[p]):
                cp.wait()

            @pl.when(i > 0)
            def _():
                for cp in ocps(bb - 1, ys[1 - p], os_[1 - p]):
                    cp.wait()

            @pl.when(i + 1 < spw)
            def _():
                pltpu.sync_copy(
                    idx_hbm.at[pl.ds((bb + 1) * _RPAD, _RPAD)], idxs[1 - p])
                for cp in gcps(idxs[1 - p], ys[1 - p], gs[1 - p]):
                    cp.start()

            def t_acc(tt, acc):
                pe4 = [pe_v[tt, pl.ds(k * 16, 16)] for k in range(4)]

                def v_acc(vv, acc2):
                    r = tt * _V + vv
                    out = list(acc2)
                    for k in range(4):
                        x = y_v[r, pl.ds(k * 16, 16)] + pe4[k]
                        y_v[r, pl.ds(k * 16, 16)] = x
                        out[k] = out[k] + x
                        out[4 + k] = out[4 + k] + x * x
                    return tuple(out)

                return lax.fori_loop(0, _V, v_acc, acc)

            acc = lax.fori_loop(0, _T, t_acc, (zero16,) * 8)
            s16 = acc[0] + acc[1] + acc[2] + acc[3]
            q16 = acc[4] + acc[5] + acc[6] + acc[7]
            mean = jnp.sum(s16) * (1.0 / _N_ELT)
            var = jnp.sum(q16) * (1.0 / _N_ELT) - mean * mean
            ve = var + _EPS
            bits = lax.bitcast_convert_type(ve, jnp.int32)
            bits = 0x5F3759DF - lax.shift_right_arithmetic(bits, 1)
            r0 = lax.bitcast_convert_type(bits, jnp.float32)
            for _ in range(4):
                r0 = r0 * (1.5 - 0.5 * ve * r0 * r0)
            inv16 = jnp.full((16,), r0, jnp.float32)
            mv16 = jnp.full((16,), mean * r0, jnp.float32)

            def r_norm(r, c):
                for k in range(4):
                    y_v[r, pl.ds(k * 16, 16)] = (
                        y_v[r, pl.ds(k * 16, 16)] * inv16 - mv16)
                return c

            lax.fori_loop(0, _ROWS, r_norm, 0)

            for cp in ocps(bb, y_v, os_[p]):
                cp.start()

        def body2(ii, carry):
            process(2 * ii, 0)
            process(2 * ii + 1, 1)
            return carry

        pltpu.sync_copy(idx_hbm.at[pl.ds(base * _RPAD, _RPAD)], idx0)
        for cp in gcps(idx0, y0, g0):
            cp.start()
        lax.fori_loop(0, spw // 2, body2, 0)
        for cp in ocps(base + spw - 1, y1, o1):
            cp.wait()

    return sc_fused


_sc_fused_cache = []


def _sc_fused(idx, table, pe):
    if not _sc_fused_cache:
        _sc_fused_cache.append(_sc_fused_fn())
    return _sc_fused_cache[0](idx, table, pe)


def kernel(diag_seq, proc_seq, drug_seq, delta_t, service, admtype, insur,
           marit, seq_length, token_table, pe_dt, pe_pos):
    b, t = delta_t.shape
    vocab, e = token_table.shape

    dt = delta_t / 15.0
    len_mask = jnp.arange(t)[None, :] < seq_length[:, None]
    dt = jnp.cumsum(jnp.round(dt), axis=1) * len_mask.astype(dt.dtype)
    dt_idx = jnp.clip(dt.astype(jnp.int32), 0, pe_dt.shape[0] - 1)

    tok34 = jnp.concatenate(
        [diag_seq, proc_seq, drug_seq, service, admtype,
         insur[..., None], marit[..., None]], axis=2)
    idx35 = jnp.concatenate([tok34, (dt_idx + vocab)[..., None]], axis=2)
    idx = jnp.pad(idx35.reshape(b, _ROWS), ((0, 0), (0, _RPAD - _ROWS)))
    idx = idx.reshape(-1)

    table64 = jnp.concatenate([token_table, pe_dt], axis=0)
    pe24 = jnp.pad(pe_pos[:t], ((0, 4), (0, 0)))

    return _sc_fused(idx, table64, pe24)
